# baseline (speedup 1.0000x reference)
import functools

import jax
import jax.numpy as jnp
from jax import lax
from jax.experimental import pallas as pl
from jax.experimental.pallas import tpu as pltpu
from jax.experimental.pallas import tpu_sc as plsc

NC = 2
NS = 16
LANES = 16
BLKR = 256


def _sc_mesh():
    return plsc.VectorSubcoreMesh(
        core_axis_name="c", subcore_axis_name="s", num_cores=NC, num_subcores=NS
    )


def _make_deg(n_pad, ch):
    half = ch // 2

    def body(row_hbm, deg_out, row_v, deg_v):
        c = lax.axis_index("c")
        s = lax.axis_index("s")
        wid = c * NS + s
        pltpu.sync_copy(row_hbm.at[wid], row_v)

        def zero(i, carry):
            deg_v[pl.ds(i * LANES, LANES)] = jnp.zeros((LANES,), jnp.float32)
            return carry

        lax.fori_loop(0, n_pad // LANES, zero, 0)
        ones = jnp.ones((LANES,), jnp.float32)

        def scat(i, carry):
            j = i // (128 // LANES)
            k = i % (128 // LANES)
            idx = row_v[j, pl.ds(k * LANES, LANES)]
            plsc.addupdate_scatter(deg_v, [idx], ones)
            return carry

        lax.fori_loop(0, half * (128 // LANES), scat, 0)
        pltpu.sync_copy(deg_v, deg_out.at[wid])

    return pl.kernel(
        body,
        out_type=jax.ShapeDtypeStruct((NC * NS, n_pad), jnp.float32),
        mesh=_sc_mesh(),
        scratch_types=[
            pltpu.VMEM((half, 128), jnp.int32),
            pltpu.VMEM((n_pad,), jnp.float32),
        ],
        compiler_params=pltpu.CompilerParams(needs_layout_passes=False),
    )


def _make_conv(n_pad, ch, dh, feature_split):
    rows_tile = n_pad // NS
    nzc = rows_tile // 128
    vpr = dh // LANES
    ch_tile = ch if feature_split else ch // 2
    nbi = 3
    gch = ch_tile // nbi

    def body(tab_hbm, row_hbm, col_hbm, y_hbm, ri, ci, buf, acc, sem, semi, sems):
        c = lax.axis_index("c")
        s = lax.axis_index("s")
        if feature_split:
            def rsrc(g):
                return row_hbm.at[c, s, g]

            def csrc(g):
                return col_hbm.at[s, g]

        else:
            tid = c * NS + s

            def rsrc(g):
                return row_hbm.at[tid, g]

            def csrc(g):
                return col_hbm.at[tid, g]

        def load_idx(g, slot):
            pltpu.async_copy(rsrc(g), ri.at[slot], semi)
            pltpu.async_copy(csrc(g), ci.at[slot], semi)

        def wait_idx(g, slot):
            pltpu.make_async_copy(rsrc(g), ri.at[slot], semi).wait()
            pltpu.make_async_copy(csrc(g), ci.at[slot], semi).wait()

        load_idx(0, 0)

        def zbuf(i, carry):
            j = i // vpr
            k = i % vpr
            z = jnp.zeros((LANES,), jnp.float32)
            buf[0, j, pl.ds(k * LANES, LANES)] = z
            buf[1, j, pl.ds(k * LANES, LANES)] = z
            return carry

        lax.fori_loop(0, 128 * vpr, zbuf, 0)

        def zacc(q, carry):
            pltpu.sync_copy(buf.at[0], acc.at[pl.ds(s * rows_tile + q * 128, 128)])
            return carry

        lax.fori_loop(0, nzc, zacc, 0)
        plsc.subcore_barrier()

        def gather(slot, j, b):
            return pltpu.async_copy(tab_hbm.at[ri.at[slot, j]], buf.at[b], sem)

        def wait_gather(slot, j, b):
            pltpu.make_async_copy(
                tab_hbm.at[ri.at[slot, j]], buf.at[b], sem
            ).wait()

        def scat(slot, j, b):
            pltpu.async_copy(buf.at[b], acc.at[ci.at[slot, j]], sems, add=True)

        def wsct(b):
            pltpu.make_async_copy(buf.at[b], acc.at[ci.at[0, 0]], sems).wait()

        wait_idx(0, 0)
        scat(0, 0, 1)
        gather(0, 0, 0)
        for g in range(nbi):
            sl = g % 2
            if g + 1 < nbi:
                load_idx(g + 1, 1 - sl)

            def pair(i, carry):
                j0 = 2 * i
                j1 = j0 + 1
                wait_gather(sl, j0, 0)
                wsct(1)
                gather(sl, j1, 1)
                scat(sl, j0, 0)
                wait_gather(sl, j1, 1)
                wsct(0)
                gather(sl, j1 + 1, 0)
                scat(sl, j1, 1)
                return carry

            lax.fori_loop(0, gch // 2 - 1, pair, 0)
            wait_gather(sl, gch - 2, 0)
            wsct(1)
            gather(sl, gch - 1, 1)
            scat(sl, gch - 2, 0)
            wait_gather(sl, gch - 1, 1)
            wsct(0)
            if g + 1 < nbi:
                wait_idx(g + 1, 1 - sl)
                gather(1 - sl, 0, 0)
            scat(sl, gch - 1, 1)
        wsct(1)
        plsc.subcore_barrier()

        for q in range(nzc):
            b = q % 2
            base = s * rows_tile + q * 128
            if q >= 2:
                pb = s * rows_tile + (q - 2) * 128
                pltpu.make_async_copy(
                    buf.at[b], y_hbm.at[c, pl.ds(pb, 128)], sem
                ).wait()
            pltpu.sync_copy(acc.at[pl.ds(base, 128)], buf.at[b])
            pltpu.async_copy(buf.at[b], y_hbm.at[c, pl.ds(base, 128)], sem)
        for q in range(max(nzc - 2, 0), nzc):
            base = s * rows_tile + q * 128
            pltpu.make_async_copy(
                buf.at[q % 2], y_hbm.at[c, pl.ds(base, 128)], sem
            ).wait()

    return pl.kernel(
        body,
        out_type=jax.ShapeDtypeStruct((NC, n_pad, dh), jnp.float32),
        mesh=_sc_mesh(),
        scratch_types=[
            pltpu.VMEM((2, gch, 128), jnp.int32),
            pltpu.VMEM((2, gch, 128), jnp.int32),
            pltpu.VMEM((2, 128, dh), jnp.float32),
            pltpu.VMEM_SHARED((n_pad, dh), jnp.float32),
            pltpu.SemaphoreType.DMA,
            pltpu.SemaphoreType.DMA,
            pltpu.SemaphoreType.DMA,
        ],
        compiler_params=pltpu.CompilerParams(needs_layout_passes=False),
    )


def _scale_body(deg_ref, x_ref, xs_ref, dinv_ref):
    deg = jnp.sum(deg_ref[...], axis=0)
    dinv = jnp.where(deg > 0.0, lax.rsqrt(deg), 0.0)
    xs_ref[...] = x_ref[...] * dinv[:, None]
    dinv_ref[...] = dinv[:, None]


def _mlp_body(y_ref, dinv_ref, w1_ref, b1_ref, w2_ref, g_ref):
    d = dinv_ref[...]
    h = jnp.dot(y_ref[0] * d, w1_ref[0], preferred_element_type=jnp.float32)
    h = h + jnp.dot(y_ref[1] * d, w1_ref[1], preferred_element_type=jnp.float32)
    h = jnp.maximum(h + b1_ref[...], 0.0)
    g_ref[...] = jnp.dot(h, w2_ref[...], preferred_element_type=jnp.float32) * d


def _out_body(y_ref, dinv_ref, b2_ref, o_ref):
    z = (y_ref[0] + y_ref[1]) * dinv_ref[...]
    z = z + b2_ref[...]
    m = jnp.max(z, axis=1, keepdims=True)
    e = jnp.exp(z - m)
    o_ref[...] = (z - m) - jnp.log(jnp.sum(e, axis=1, keepdims=True))


def kernel(x, edge_index, W1, b1, W2, b2):
    n, d_in = x.shape
    d_hid = W1.shape[1]
    d_out = W2.shape[1]
    dh1 = d_in // NC
    dh2 = d_out // NC

    row = edge_index[0].astype(jnp.int32)
    col = edge_index[1].astype(jnp.int32)
    loop = jnp.arange(n, dtype=jnp.int32)
    row_e = jnp.where(row == col, n, row)
    row_all = jnp.concatenate([row_e, loop])
    col_all = jnp.concatenate([col, loop])
    e_tot = row_all.shape[0]

    n_pad = -(-(n + 1) // (NS * 128)) * (NS * 128)
    ch = -(-e_tot // (NS * 128))
    ch = -(-ch // 12) * 12
    e_pad = NS * ch * 128
    row_all = jnp.concatenate([row_all, jnp.full((e_pad - e_tot,), n, jnp.int32)])
    col_all = jnp.concatenate([col_all, jnp.full((e_pad - e_tot,), n, jnp.int32)])
    row_c1 = row_all.reshape(NS, 3, ch // 3, 128)
    col_c1 = col_all.reshape(NS, 3, ch // 3, 128)
    row_c2 = row_all.reshape(NC * NS, 3, ch // 6, 128)
    col_c2 = col_all.reshape(NC * NS, 3, ch // 6, 128)

    x_pad = jnp.zeros((n_pad, d_in), x.dtype).at[:n].set(x)

    deg_parts = _make_deg(n_pad, ch)(row_all.reshape(NC * NS, ch // 2, 128))

    nblk = n_pad // BLKR
    xs, dinv = pl.pallas_call(
        _scale_body,
        grid=(NC, nblk),
        in_specs=[
            pl.BlockSpec((NC * NS, BLKR), lambda h, r: (0, r)),
            pl.BlockSpec((BLKR, dh1), lambda h, r: (r, h)),
        ],
        out_specs=[
            pl.BlockSpec((BLKR, dh1), lambda h, r: (h * nblk + r, 0)),
            pl.BlockSpec((BLKR, 1), lambda h, r: (r, 0)),
        ],
        out_shape=[
            jax.ShapeDtypeStruct((NC * n_pad, dh1), jnp.float32),
            jax.ShapeDtypeStruct((n_pad, 1), jnp.float32),
        ],
    )(deg_parts, x_pad)

    row_c1_off = jnp.stack([row_c1, row_c1 + n_pad])
    y1 = _make_conv(n_pad, ch, dh1, True)(xs, row_c1_off, col_c1)

    g = pl.pallas_call(
        _mlp_body,
        grid=(nblk,),
        in_specs=[
            pl.BlockSpec((NC, BLKR, dh1), lambda r: (0, r, 0)),
            pl.BlockSpec((BLKR, 1), lambda r: (r, 0)),
            pl.BlockSpec((NC, dh1, d_hid), lambda r: (0, 0, 0)),
            pl.BlockSpec((1, d_hid), lambda r: (0, 0)),
            pl.BlockSpec((d_hid, d_out), lambda r: (0, 0)),
        ],
        out_specs=pl.BlockSpec((BLKR, d_out), lambda r: (r, 0)),
        out_shape=jax.ShapeDtypeStruct((n_pad, d_out), jnp.float32),
    )(y1, dinv, W1.reshape(NC, dh1, d_hid), b1.reshape(1, d_hid), W2)

    y2 = _make_conv(n_pad, ch, d_out, False)(g, row_c2, col_c2)

    out = pl.pallas_call(
        _out_body,
        grid=(nblk,),
        in_specs=[
            pl.BlockSpec((NC, BLKR, d_out), lambda r: (0, r, 0)),
            pl.BlockSpec((BLKR, 1), lambda r: (r, 0)),
            pl.BlockSpec((1, d_out), lambda r: (0, 0)),
        ],
        out_specs=pl.BlockSpec((BLKR, d_out), lambda r: (r, 0)),
        out_shape=jax.ShapeDtypeStruct((n_pad, d_out), jnp.float32),
    )(y2, dinv, b2.reshape(1, d_out))

    return out[:n]


# speedup vs baseline: 1.1255x; 1.1255x over previous
import functools

import jax
import jax.numpy as jnp
from jax import lax
from jax.experimental import pallas as pl
from jax.experimental.pallas import tpu as pltpu
from jax.experimental.pallas import tpu_sc as plsc

NC = 2
NS = 16
LANES = 16
BLKR = 1024


def _sc_mesh():
    return plsc.VectorSubcoreMesh(
        core_axis_name="c", subcore_axis_name="s", num_cores=NC, num_subcores=NS
    )


def _make_deg(n_pad, ch):
    half = ch // 2

    def body(row_hbm, deg_out, row_v, deg_v):
        c = lax.axis_index("c")
        s = lax.axis_index("s")
        wid = c * NS + s
        pltpu.sync_copy(row_hbm.at[wid], row_v)

        def zero(i, carry):
            deg_v[pl.ds(i * LANES, LANES)] = jnp.zeros((LANES,), jnp.float32)
            return carry

        lax.fori_loop(0, n_pad // LANES, zero, 0)
        ones = jnp.ones((LANES,), jnp.float32)

        def scat(i, carry):
            j = i // (128 // LANES)
            k = i % (128 // LANES)
            idx = row_v[j, pl.ds(k * LANES, LANES)]
            plsc.addupdate_scatter(deg_v, [idx], ones)
            return carry

        lax.fori_loop(0, half * (128 // LANES), scat, 0)
        pltpu.sync_copy(deg_v, deg_out.at[wid])

    return pl.kernel(
        body,
        out_type=jax.ShapeDtypeStruct((NC * NS, n_pad), jnp.float32),
        mesh=_sc_mesh(),
        scratch_types=[
            pltpu.VMEM((half, 128), jnp.int32),
            pltpu.VMEM((n_pad,), jnp.float32),
        ],
        compiler_params=pltpu.CompilerParams(needs_layout_passes=False),
    )


def _make_conv(n_pad, ch, dh, feature_split):
    rows_tile = n_pad // NS
    nzc = rows_tile // 128
    vpr = dh // LANES
    ch_tile = ch if feature_split else ch // 2
    nbi = 3
    gch = ch_tile // nbi

    def body(tab_hbm, row_hbm, col_hbm, y_hbm, ri, ci, buf, acc, sem, semi, sems):
        c = lax.axis_index("c")
        s = lax.axis_index("s")
        if feature_split:
            def rsrc(g):
                return row_hbm.at[c, s, g]

            def csrc(g):
                return col_hbm.at[s, g]

        else:
            tid = c * NS + s

            def rsrc(g):
                return row_hbm.at[tid, g]

            def csrc(g):
                return col_hbm.at[tid, g]

        def load_idx(g, slot):
            pltpu.async_copy(rsrc(g), ri.at[slot], semi)
            pltpu.async_copy(csrc(g), ci.at[slot], semi)

        def wait_idx(g, slot):
            pltpu.make_async_copy(rsrc(g), ri.at[slot], semi).wait()
            pltpu.make_async_copy(csrc(g), ci.at[slot], semi).wait()

        load_idx(0, 0)

        def zbuf(i, carry):
            j = i // vpr
            k = i % vpr
            z = jnp.zeros((LANES,), jnp.float32)
            buf[0, j, pl.ds(k * LANES, LANES)] = z
            buf[1, j, pl.ds(k * LANES, LANES)] = z
            return carry

        lax.fori_loop(0, 128 * vpr, zbuf, 0)

        def zacc(q, carry):
            pltpu.sync_copy(buf.at[0], acc.at[pl.ds(s * rows_tile + q * 128, 128)])
            return carry

        lax.fori_loop(0, nzc, zacc, 0)
        plsc.subcore_barrier()

        def gather(slot, j, b):
            return pltpu.async_copy(tab_hbm.at[ri.at[slot, j]], buf.at[b], sem)

        def wait_gather(slot, j, b):
            pltpu.make_async_copy(
                tab_hbm.at[ri.at[slot, j]], buf.at[b], sem
            ).wait()

        def scat(slot, j, b):
            pltpu.async_copy(buf.at[b], acc.at[ci.at[slot, j]], sems, add=True)

        def wsct(b):
            pltpu.make_async_copy(buf.at[b], acc.at[ci.at[0, 0]], sems).wait()

        wait_idx(0, 0)
        scat(0, 0, 1)
        gather(0, 0, 0)
        for g in range(nbi):
            sl = g % 2
            if g + 1 < nbi:
                load_idx(g + 1, 1 - sl)

            def pair(i, carry):
                j0 = 2 * i
                j1 = j0 + 1
                wait_gather(sl, j0, 0)
                wsct(1)
                gather(sl, j1, 1)
                scat(sl, j0, 0)
                wait_gather(sl, j1, 1)
                wsct(0)
                gather(sl, j1 + 1, 0)
                scat(sl, j1, 1)
                return carry

            lax.fori_loop(0, gch // 2 - 1, pair, 0)
            wait_gather(sl, gch - 2, 0)
            wsct(1)
            gather(sl, gch - 1, 1)
            scat(sl, gch - 2, 0)
            wait_gather(sl, gch - 1, 1)
            wsct(0)
            if g + 1 < nbi:
                wait_idx(g + 1, 1 - sl)
                gather(1 - sl, 0, 0)
            scat(sl, gch - 1, 1)
        wsct(1)
        plsc.subcore_barrier()

        for q in range(nzc):
            b = q % 2
            base = s * rows_tile + q * 128
            if q >= 2:
                pb = s * rows_tile + (q - 2) * 128
                pltpu.make_async_copy(
                    buf.at[b], y_hbm.at[c, pl.ds(pb, 128)], sem
                ).wait()
            pltpu.sync_copy(acc.at[pl.ds(base, 128)], buf.at[b])
            pltpu.async_copy(buf.at[b], y_hbm.at[c, pl.ds(base, 128)], sem)
        for q in range(max(nzc - 2, 0), nzc):
            base = s * rows_tile + q * 128
            pltpu.make_async_copy(
                buf.at[q % 2], y_hbm.at[c, pl.ds(base, 128)], sem
            ).wait()

    return pl.kernel(
        body,
        out_type=jax.ShapeDtypeStruct((NC, n_pad, dh), jnp.float32),
        mesh=_sc_mesh(),
        scratch_types=[
            pltpu.VMEM((2, gch, 128), jnp.int32),
            pltpu.VMEM((2, gch, 128), jnp.int32),
            pltpu.VMEM((2, 128, dh), jnp.float32),
            pltpu.VMEM_SHARED((n_pad, dh), jnp.float32),
            pltpu.SemaphoreType.DMA,
            pltpu.SemaphoreType.DMA,
            pltpu.SemaphoreType.DMA,
        ],
        compiler_params=pltpu.CompilerParams(needs_layout_passes=False),
    )


def _scale_body(deg_ref, x_ref, xs_ref, dinv_ref):
    deg = jnp.sum(deg_ref[...], axis=0)
    dinv = jnp.where(deg > 0.0, lax.rsqrt(deg), 0.0)
    xs_ref[...] = x_ref[...] * dinv[:, None]
    dinv_ref[...] = dinv[:, None]


def _mlp_body(y_ref, dinv_ref, w1_ref, b1_ref, w2_ref, g_ref):
    d = dinv_ref[...]
    h = jnp.dot(y_ref[0] * d, w1_ref[0], preferred_element_type=jnp.float32)
    h = h + jnp.dot(y_ref[1] * d, w1_ref[1], preferred_element_type=jnp.float32)
    h = jnp.maximum(h + b1_ref[...], 0.0)
    g_ref[...] = jnp.dot(h, w2_ref[...], preferred_element_type=jnp.float32) * d


def _out_body(y_ref, dinv_ref, b2_ref, o_ref):
    z = (y_ref[0] + y_ref[1]) * dinv_ref[...]
    z = z + b2_ref[...]
    m = jnp.max(z, axis=1, keepdims=True)
    e = jnp.exp(z - m)
    o_ref[...] = (z - m) - jnp.log(jnp.sum(e, axis=1, keepdims=True))


def kernel(x, edge_index, W1, b1, W2, b2):
    n, d_in = x.shape
    d_hid = W1.shape[1]
    d_out = W2.shape[1]
    dh1 = d_in // NC
    dh2 = d_out // NC

    row = edge_index[0].astype(jnp.int32)
    col = edge_index[1].astype(jnp.int32)
    loop = jnp.arange(n, dtype=jnp.int32)
    row_e = jnp.where(row == col, n, row)
    row_all = jnp.concatenate([row_e, loop])
    col_all = jnp.concatenate([col, loop])
    e_tot = row_all.shape[0]

    n_pad = -(-(n + 1) // (NS * 128)) * (NS * 128)
    ch = -(-e_tot // (NS * 128))
    ch = -(-ch // 12) * 12
    e_pad = NS * ch * 128
    row_all = jnp.concatenate([row_all, jnp.full((e_pad - e_tot,), n, jnp.int32)])
    col_all = jnp.concatenate([col_all, jnp.full((e_pad - e_tot,), n, jnp.int32)])
    row_c1 = row_all.reshape(NS, 3, ch // 3, 128)
    col_c1 = col_all.reshape(NS, 3, ch // 3, 128)
    row_c2 = row_all.reshape(NC * NS, 3, ch // 6, 128)
    col_c2 = col_all.reshape(NC * NS, 3, ch // 6, 128)

    x_pad = jnp.zeros((n_pad, d_in), x.dtype).at[:n].set(x)

    deg_parts = _make_deg(n_pad, ch)(row_all.reshape(NC * NS, ch // 2, 128))

    nblk = n_pad // BLKR
    xs, dinv = pl.pallas_call(
        _scale_body,
        grid=(NC, nblk),
        in_specs=[
            pl.BlockSpec((NC * NS, BLKR), lambda h, r: (0, r)),
            pl.BlockSpec((BLKR, dh1), lambda h, r: (r, h)),
        ],
        out_specs=[
            pl.BlockSpec((BLKR, dh1), lambda h, r: (h * nblk + r, 0)),
            pl.BlockSpec((BLKR, 1), lambda h, r: (r, 0)),
        ],
        out_shape=[
            jax.ShapeDtypeStruct((NC * n_pad, dh1), jnp.float32),
            jax.ShapeDtypeStruct((n_pad, 1), jnp.float32),
        ],
    )(deg_parts, x_pad)

    row_c1_off = jnp.stack([row_c1, row_c1 + n_pad])
    y1 = _make_conv(n_pad, ch, dh1, True)(xs, row_c1_off, col_c1)

    g = pl.pallas_call(
        _mlp_body,
        grid=(nblk,),
        in_specs=[
            pl.BlockSpec((NC, BLKR, dh1), lambda r: (0, r, 0)),
            pl.BlockSpec((BLKR, 1), lambda r: (r, 0)),
            pl.BlockSpec((NC, dh1, d_hid), lambda r: (0, 0, 0)),
            pl.BlockSpec((1, d_hid), lambda r: (0, 0)),
            pl.BlockSpec((d_hid, d_out), lambda r: (0, 0)),
        ],
        out_specs=pl.BlockSpec((BLKR, d_out), lambda r: (r, 0)),
        out_shape=jax.ShapeDtypeStruct((n_pad, d_out), jnp.float32),
    )(y1, dinv, W1.reshape(NC, dh1, d_hid), b1.reshape(1, d_hid), W2)

    y2 = _make_conv(n_pad, ch, d_out, False)(g, row_c2, col_c2)

    out = pl.pallas_call(
        _out_body,
        grid=(nblk,),
        in_specs=[
            pl.BlockSpec((NC, BLKR, d_out), lambda r: (0, r, 0)),
            pl.BlockSpec((BLKR, 1), lambda r: (r, 0)),
            pl.BlockSpec((1, d_out), lambda r: (0, 0)),
        ],
        out_specs=pl.BlockSpec((BLKR, d_out), lambda r: (r, 0)),
        out_shape=jax.ShapeDtypeStruct((n_pad, d_out), jnp.float32),
    )(y2, dinv, b2.reshape(1, d_out))

    return out[:n]


# speedup vs baseline: 1.1469x; 1.0190x over previous
import functools

import jax
import jax.numpy as jnp
from jax import lax
from jax.experimental import pallas as pl
from jax.experimental.pallas import tpu as pltpu
from jax.experimental.pallas import tpu_sc as plsc

NC = 2
NS = 16
LANES = 16
BLKR = 2048


def _sc_mesh():
    return plsc.VectorSubcoreMesh(
        core_axis_name="c", subcore_axis_name="s", num_cores=NC, num_subcores=NS
    )


def _make_deg(n_pad, ch):
    half = ch // 2

    def body(row_hbm, deg_out, row_v, deg_v):
        c = lax.axis_index("c")
        s = lax.axis_index("s")
        wid = c * NS + s
        pltpu.sync_copy(row_hbm.at[wid], row_v)

        def zero(i, carry):
            deg_v[pl.ds(i * LANES, LANES)] = jnp.zeros((LANES,), jnp.float32)
            return carry

        lax.fori_loop(0, n_pad // LANES, zero, 0)
        ones = jnp.ones((LANES,), jnp.float32)

        def scat(i, carry):
            j = i // (128 // LANES)
            k = i % (128 // LANES)
            idx = row_v[j, pl.ds(k * LANES, LANES)]
            plsc.addupdate_scatter(deg_v, [idx], ones)
            return carry

        lax.fori_loop(0, half * (128 // LANES), scat, 0)
        pltpu.sync_copy(deg_v, deg_out.at[wid])

    return pl.kernel(
        body,
        out_type=jax.ShapeDtypeStruct((NC * NS, n_pad), jnp.float32),
        mesh=_sc_mesh(),
        scratch_types=[
            pltpu.VMEM((half, 128), jnp.int32),
            pltpu.VMEM((n_pad,), jnp.float32),
        ],
        compiler_params=pltpu.CompilerParams(needs_layout_passes=False),
    )


def _make_conv(n_pad, ch, dh, feature_split):
    rows_tile = n_pad // NS
    nzc = rows_tile // 128
    vpr = dh // LANES
    ch_tile = ch if feature_split else ch // 2
    nbi = 3
    gch = ch_tile // nbi

    def body(tab_hbm, row_hbm, col_hbm, y_hbm, ri, ci, buf, acc, sem, semi, sems):
        c = lax.axis_index("c")
        s = lax.axis_index("s")
        if feature_split:
            def rsrc(g):
                return row_hbm.at[c, s, g]

            def csrc(g):
                return col_hbm.at[s, g]

        else:
            tid = c * NS + s

            def rsrc(g):
                return row_hbm.at[tid, g]

            def csrc(g):
                return col_hbm.at[tid, g]

        def load_idx(g, slot):
            pltpu.async_copy(rsrc(g), ri.at[slot], semi)
            pltpu.async_copy(csrc(g), ci.at[slot], semi)

        def wait_idx(g, slot):
            pltpu.make_async_copy(rsrc(g), ri.at[slot], semi).wait()
            pltpu.make_async_copy(csrc(g), ci.at[slot], semi).wait()

        load_idx(0, 0)

        def zbuf(i, carry):
            j = i // vpr
            k = i % vpr
            z = jnp.zeros((LANES,), jnp.float32)
            buf[0, j, pl.ds(k * LANES, LANES)] = z
            buf[1, j, pl.ds(k * LANES, LANES)] = z
            return carry

        lax.fori_loop(0, 128 * vpr, zbuf, 0)

        def zacc(q, carry):
            pltpu.sync_copy(buf.at[0], acc.at[pl.ds(s * rows_tile + q * 128, 128)])
            return carry

        lax.fori_loop(0, nzc, zacc, 0)
        plsc.subcore_barrier()

        def gather(slot, j, b):
            return pltpu.async_copy(tab_hbm.at[ri.at[slot, j]], buf.at[b], sem)

        def wait_gather(slot, j, b):
            pltpu.make_async_copy(
                tab_hbm.at[ri.at[slot, j]], buf.at[b], sem
            ).wait()

        def scat(slot, j, b):
            pltpu.async_copy(buf.at[b], acc.at[ci.at[slot, j]], sems, add=True)

        def wsct(b):
            pltpu.make_async_copy(buf.at[b], acc.at[ci.at[0, 0]], sems).wait()

        wait_idx(0, 0)
        scat(0, 0, 1)
        gather(0, 0, 0)
        for g in range(nbi):
            sl = g % 2
            if g + 1 < nbi:
                load_idx(g + 1, 1 - sl)

            def pair(i, carry):
                j0 = 2 * i
                j1 = j0 + 1
                wait_gather(sl, j0, 0)
                wsct(1)
                gather(sl, j1, 1)
                scat(sl, j0, 0)
                wait_gather(sl, j1, 1)
                wsct(0)
                gather(sl, j1 + 1, 0)
                scat(sl, j1, 1)
                return carry

            lax.fori_loop(0, gch // 2 - 1, pair, 0)
            wait_gather(sl, gch - 2, 0)
            wsct(1)
            gather(sl, gch - 1, 1)
            scat(sl, gch - 2, 0)
            wait_gather(sl, gch - 1, 1)
            wsct(0)
            if g + 1 < nbi:
                wait_idx(g + 1, 1 - sl)
                gather(1 - sl, 0, 0)
            scat(sl, gch - 1, 1)
        wsct(1)
        plsc.subcore_barrier()

        for q in range(nzc):
            b = q % 2
            base = s * rows_tile + q * 128
            if q >= 2:
                pb = s * rows_tile + (q - 2) * 128
                pltpu.make_async_copy(
                    buf.at[b], y_hbm.at[c, pl.ds(pb, 128)], sem
                ).wait()
            pltpu.sync_copy(acc.at[pl.ds(base, 128)], buf.at[b])
            pltpu.async_copy(buf.at[b], y_hbm.at[c, pl.ds(base, 128)], sem)
        for q in range(max(nzc - 2, 0), nzc):
            base = s * rows_tile + q * 128
            pltpu.make_async_copy(
                buf.at[q % 2], y_hbm.at[c, pl.ds(base, 128)], sem
            ).wait()

    return pl.kernel(
        body,
        out_type=jax.ShapeDtypeStruct((NC, n_pad, dh), jnp.float32),
        mesh=_sc_mesh(),
        scratch_types=[
            pltpu.VMEM((2, gch, 128), jnp.int32),
            pltpu.VMEM((2, gch, 128), jnp.int32),
            pltpu.VMEM((2, 128, dh), jnp.float32),
            pltpu.VMEM_SHARED((n_pad, dh), jnp.float32),
            pltpu.SemaphoreType.DMA,
            pltpu.SemaphoreType.DMA,
            pltpu.SemaphoreType.DMA,
        ],
        compiler_params=pltpu.CompilerParams(needs_layout_passes=False),
    )


def _scale_body(deg_ref, x_ref, xs_ref, dinv_ref):
    deg = jnp.sum(deg_ref[...], axis=0)
    dinv = jnp.where(deg > 0.0, lax.rsqrt(deg), 0.0)
    xs_ref[...] = x_ref[...] * dinv[:, None]
    dinv_ref[...] = dinv[:, None]


def _mlp_body(y_ref, dinv_ref, w1_ref, b1_ref, w2_ref, g_ref):
    d = dinv_ref[...]
    h = jnp.dot(y_ref[0] * d, w1_ref[0], preferred_element_type=jnp.float32)
    h = h + jnp.dot(y_ref[1] * d, w1_ref[1], preferred_element_type=jnp.float32)
    h = jnp.maximum(h + b1_ref[...], 0.0)
    g_ref[...] = jnp.dot(h, w2_ref[...], preferred_element_type=jnp.float32) * d


def _out_body(y_ref, dinv_ref, b2_ref, o_ref):
    z = (y_ref[0] + y_ref[1]) * dinv_ref[...]
    z = z + b2_ref[...]
    m = jnp.max(z, axis=1, keepdims=True)
    e = jnp.exp(z - m)
    o_ref[...] = (z - m) - jnp.log(jnp.sum(e, axis=1, keepdims=True))


def kernel(x, edge_index, W1, b1, W2, b2):
    n, d_in = x.shape
    d_hid = W1.shape[1]
    d_out = W2.shape[1]
    dh1 = d_in // NC
    dh2 = d_out // NC

    row = edge_index[0].astype(jnp.int32)
    col = edge_index[1].astype(jnp.int32)
    loop = jnp.arange(n, dtype=jnp.int32)
    row_e = jnp.where(row == col, n, row)
    row_all = jnp.concatenate([row_e, loop])
    col_all = jnp.concatenate([col, loop])
    e_tot = row_all.shape[0]

    n_pad = -(-(n + 1) // (NS * 128)) * (NS * 128)
    ch = -(-e_tot // (NS * 128))
    ch = -(-ch // 12) * 12
    e_pad = NS * ch * 128
    row_all = jnp.concatenate([row_all, jnp.full((e_pad - e_tot,), n, jnp.int32)])
    col_all = jnp.concatenate([col_all, jnp.full((e_pad - e_tot,), n, jnp.int32)])
    row_c1 = row_all.reshape(NS, 3, ch // 3, 128)
    col_c1 = col_all.reshape(NS, 3, ch // 3, 128)
    row_c2 = row_all.reshape(NC * NS, 3, ch // 6, 128)
    col_c2 = col_all.reshape(NC * NS, 3, ch // 6, 128)

    x_pad = jnp.zeros((n_pad, d_in), x.dtype).at[:n].set(x)

    deg_parts = _make_deg(n_pad, ch)(row_all.reshape(NC * NS, ch // 2, 128))

    nblk = n_pad // BLKR
    xs, dinv = pl.pallas_call(
        _scale_body,
        grid=(NC, nblk),
        in_specs=[
            pl.BlockSpec((NC * NS, BLKR), lambda h, r: (0, r)),
            pl.BlockSpec((BLKR, dh1), lambda h, r: (r, h)),
        ],
        out_specs=[
            pl.BlockSpec((BLKR, dh1), lambda h, r: (h * nblk + r, 0)),
            pl.BlockSpec((BLKR, 1), lambda h, r: (r, 0)),
        ],
        out_shape=[
            jax.ShapeDtypeStruct((NC * n_pad, dh1), jnp.float32),
            jax.ShapeDtypeStruct((n_pad, 1), jnp.float32),
        ],
    )(deg_parts, x_pad)

    row_c1_off = jnp.stack([row_c1, row_c1 + n_pad])
    y1 = _make_conv(n_pad, ch, dh1, True)(xs, row_c1_off, col_c1)

    g = pl.pallas_call(
        _mlp_body,
        grid=(nblk,),
        in_specs=[
            pl.BlockSpec((NC, BLKR, dh1), lambda r: (0, r, 0)),
            pl.BlockSpec((BLKR, 1), lambda r: (r, 0)),
            pl.BlockSpec((NC, dh1, d_hid), lambda r: (0, 0, 0)),
            pl.BlockSpec((1, d_hid), lambda r: (0, 0)),
            pl.BlockSpec((d_hid, d_out), lambda r: (0, 0)),
        ],
        out_specs=pl.BlockSpec((BLKR, d_out), lambda r: (r, 0)),
        out_shape=jax.ShapeDtypeStruct((n_pad, d_out), jnp.float32),
    )(y1, dinv, W1.reshape(NC, dh1, d_hid), b1.reshape(1, d_hid), W2)

    y2 = _make_conv(n_pad, ch, d_out, False)(g, row_c2, col_c2)

    out = pl.pallas_call(
        _out_body,
        grid=(nblk,),
        in_specs=[
            pl.BlockSpec((NC, BLKR, d_out), lambda r: (0, r, 0)),
            pl.BlockSpec((BLKR, 1), lambda r: (r, 0)),
            pl.BlockSpec((1, d_out), lambda r: (0, 0)),
        ],
        out_specs=pl.BlockSpec((BLKR, d_out), lambda r: (r, 0)),
        out_shape=jax.ShapeDtypeStruct((n_pad, d_out), jnp.float32),
    )(y2, dinv, b2.reshape(1, d_out))

    return out[:n]
